# baseline (speedup 1.0000x reference)
import functools

import jax
import jax.numpy as jnp
from jax.experimental import pallas as pl
from jax.experimental.pallas import tpu as pltpu


def _build_ae(a_ref, ae_ref, *, V, W, L, TR):
    K = V * L
    Nw = W * L
    vpg = 128 // L
    a = a_ref[...].astype(jnp.bfloat16)
    ar = jnp.broadcast_to(
        a.reshape(V // vpg, vpg, 1, W), (V // vpg, vpg, L, W)
    ).reshape(K, W)
    col = jax.lax.broadcasted_iota(jnp.int32, (W, Nw), 1)
    row = jax.lax.broadcasted_iota(jnp.int32, (W, Nw), 0)
    wc = (col // 128) * vpg + (col % 128) // L
    rep = (wc == row).astype(jnp.bfloat16)
    mrow = jax.lax.broadcasted_iota(jnp.int32, (TR, Nw), 0)
    mcol = jax.lax.broadcasted_iota(jnp.int32, (TR, Nw), 1)
    mask = ((mrow % L) == (mcol % L)).astype(jnp.bfloat16)
    for t in range(K // TR):
        arl = jnp.dot(ar[t * TR:(t + 1) * TR, :], rep,
                      preferred_element_type=jnp.float32,
                      precision=jax.lax.Precision.DEFAULT)
        ae_ref[t * TR:(t + 1) * TR, :] = arl.astype(jnp.bfloat16) * mask


def _nconv_kernel(x_hbm, a_ref, o_hbm, ae_ref, x_buf, o_buf, in_sem,
                  out_sem, *, V, W, L, TR, tb, steps):
    K = V * L
    Nw = W * L
    GI = K // 128
    GO = Nw // 128
    core = pl.program_id(0)
    base = core * steps * tb

    def start_in(slot, step):
        row = base + step * tb
        for g in range(GI):
            pltpu.make_async_copy(
                x_hbm.at[pl.ds(row, tb), g],
                x_buf.at[slot, :, pl.ds(g * 128, 128)],
                in_sem.at[slot]).start()

    def wait_in(slot):
        for g in range(GI):
            pltpu.make_async_copy(
                x_hbm.at[pl.ds(0, tb), 0],
                x_buf.at[slot, :, pl.ds(0, 128)],
                in_sem.at[slot]).wait()

    def start_out(slot, step):
        row = base + step * tb
        for g in range(GO):
            pltpu.make_async_copy(
                o_buf.at[slot, :, pl.ds(g * 128, 128)],
                o_hbm.at[pl.ds(row, tb), g],
                out_sem.at[slot]).start()

    def wait_out(slot):
        for g in range(GO):
            pltpu.make_async_copy(
                o_buf.at[slot, :, pl.ds(0, 128)],
                o_hbm.at[pl.ds(0, tb), 0],
                out_sem.at[slot]).wait()

    start_in(0, 0)
    _build_ae(a_ref, ae_ref, V=V, W=W, L=L, TR=TR)
    ae = ae_ref[...]

    def body(step, _):
        cur = jax.lax.rem(step, 2)
        nxt = jax.lax.rem(step + 1, 2)

        @pl.when(step + 1 < steps)
        def _():
            start_in(nxt, step + 1)

        wait_in(cur)

        @pl.when(step >= 2)
        def _():
            wait_out(cur)

        o_buf[cur] = jnp.dot(
            x_buf[cur].astype(jnp.bfloat16), ae,
            preferred_element_type=jnp.float32,
            precision=jax.lax.Precision.DEFAULT,
        ).astype(o_buf.dtype)
        start_out(cur, step)
        return ()

    jax.lax.fori_loop(0, steps, body, ())
    if steps >= 2:
        wait_out(jax.lax.rem(steps - 2, 2))
    wait_out(jax.lax.rem(steps - 1, 2))


@jax.jit
def kernel(x, A):
    N, C, V, L = x.shape
    V2, W = A.shape
    assert V == V2
    B = N * C
    K = V * L
    Nw = W * L
    assert K % 128 == 0 and Nw % 128 == 0 and 128 % L == 0
    x6 = x.reshape(B, K // 128, 128)

    tb = min(128, B)
    nblk = B // tb
    assert nblk * tb == B
    ncores = 2 if nblk % 2 == 0 else 1
    steps = nblk // ncores
    TR = min(256, K)
    assert K % TR == 0 and TR % L == 0

    out = pl.pallas_call(
        functools.partial(_nconv_kernel, V=V, W=W, L=L, TR=TR, tb=tb,
                          steps=steps),
        out_shape=jax.ShapeDtypeStruct((B, Nw // 128, 128), x.dtype),
        grid=(ncores,),
        in_specs=[
            pl.BlockSpec(memory_space=pl.ANY),
            pl.BlockSpec((V, W), lambda i: (0, 0)),
        ],
        out_specs=pl.BlockSpec(memory_space=pl.ANY),
        scratch_shapes=[
            pltpu.VMEM((K, Nw), jnp.bfloat16),
            pltpu.VMEM((2, tb, K), jnp.float32),
            pltpu.VMEM((2, tb, Nw), jnp.float32),
            pltpu.SemaphoreType.DMA((2,)),
            pltpu.SemaphoreType.DMA((2,)),
        ],
        compiler_params=pltpu.CompilerParams(
            dimension_semantics=("parallel",),
            vmem_limit_bytes=int(56 << 20),
        ),
    )(x6, A)
    return out.reshape(N, C, W, L)


# speedup vs baseline: 24.9842x; 24.9842x over previous
import jax
import jax.numpy as jnp
from jax.experimental import pallas as pl
from jax.experimental.pallas import tpu as pltpu


def _matmul_kernel(x_ref, a_ref, o_ref):
    o_ref[...] = jnp.dot(
        x_ref[...],
        a_ref[...],
        preferred_element_type=jnp.float32,
    ).astype(o_ref.dtype)


@jax.jit
def kernel(x, A):
    N, C, V, L = x.shape
    V2, W = A.shape
    assert V == V2
    M = N * C * L

    x2 = jnp.transpose(x, (0, 1, 3, 2)).reshape(M, V)

    tr = min(4096, M)
    grid = pl.cdiv(M, tr)

    out2 = pl.pallas_call(
        _matmul_kernel,
        out_shape=jax.ShapeDtypeStruct((M, W), jnp.float32),
        grid=(grid,),
        in_specs=[
            pl.BlockSpec((tr, V), lambda i: (i, 0)),
            pl.BlockSpec((V, W), lambda i: (0, 0)),
        ],
        out_specs=pl.BlockSpec((tr, W), lambda i: (i, 0)),
        compiler_params=pltpu.CompilerParams(
            dimension_semantics=("parallel",),
            vmem_limit_bytes=int(32 << 20),
        ),
    )(x2, A)

    return out2.reshape(N, C, L, W).transpose(0, 1, 3, 2)


# speedup vs baseline: 26.4851x; 1.0601x over previous
import jax
import jax.numpy as jnp
from jax.experimental import pallas as pl
from jax.experimental.pallas import tpu as pltpu


def _matmul_kernel(x_ref, a_ref, o_ref):
    o_ref[...] = jnp.dot(
        x_ref[...],
        a_ref[...],
        preferred_element_type=jnp.float32,
    ).astype(o_ref.dtype)


@jax.jit
def kernel(x, A):
    N, C, V, L = x.shape
    V2, W = A.shape
    assert V == V2
    M = N * C * L

    x2 = jnp.transpose(x, (0, 1, 3, 2)).reshape(M, V)

    tr = min(8192, M)
    grid = pl.cdiv(M, tr)

    out2 = pl.pallas_call(
        _matmul_kernel,
        out_shape=jax.ShapeDtypeStruct((M, W), jnp.float32),
        grid=(grid,),
        in_specs=[
            pl.BlockSpec((tr, V), lambda i: (i, 0)),
            pl.BlockSpec((V, W), lambda i: (0, 0)),
        ],
        out_specs=pl.BlockSpec((tr, W), lambda i: (i, 0)),
        compiler_params=pltpu.CompilerParams(
            dimension_semantics=("parallel",),
            vmem_limit_bytes=int(32 << 20),
        ),
    )(x2, A)

    return out2.reshape(N, C, L, W).transpose(0, 1, 3, 2)


# speedup vs baseline: 26.8133x; 1.0124x over previous
import jax
import jax.numpy as jnp
from jax.experimental import pallas as pl
from jax.experimental.pallas import tpu as pltpu


def _matmul_kernel(x_ref, a_ref, o_ref):
    o_ref[...] = jnp.dot(
        x_ref[...],
        a_ref[...],
        preferred_element_type=jnp.float32,
    ).astype(o_ref.dtype)


@jax.jit
def kernel(x, A):
    N, C, V, L = x.shape
    V2, W = A.shape
    assert V == V2
    M = N * C * L

    x2 = jnp.transpose(x, (0, 1, 3, 2)).reshape(M, V)

    tr = min(6144, M)
    grid = pl.cdiv(M, tr)

    out2 = pl.pallas_call(
        _matmul_kernel,
        out_shape=jax.ShapeDtypeStruct((M, W), jnp.float32),
        grid=(grid,),
        in_specs=[
            pl.BlockSpec((tr, V), lambda i: (i, 0)),
            pl.BlockSpec((V, W), lambda i: (0, 0)),
        ],
        out_specs=pl.BlockSpec((tr, W), lambda i: (i, 0)),
        compiler_params=pltpu.CompilerParams(
            dimension_semantics=("parallel",),
            vmem_limit_bytes=int(32 << 20),
        ),
    )(x2, A)

    return out2.reshape(N, C, L, W).transpose(0, 1, 3, 2)
